# baseline (speedup 1.0000x reference)
import functools

import jax
import jax.numpy as jnp
from jax import lax
from jax.experimental import pallas as pl
from jax.experimental.pallas import tpu as pltpu
from jax.experimental.pallas import tpu_sc as plsc

VOCAB = 1000000
DIM = 64
WDIM = 128
BATCH = 16384

NUM_CORES = 2
NUM_SUBCORES = 16
NW = NUM_CORES * NUM_SUBCORES
BASE_COLS = 31232
EXTRA = 4
TAIL_START = 999936
TAIL_COLS = VOCAB - TAIL_START
CCOLS = 512
NCHUNK = 62
WINDOW = 256
NIDX_G = BATCH // 16
OUT_ROWS = BATCH + NW

_mesh = plsc.VectorSubcoreMesh(core_axis_name="c", subcore_axis_name="s")


@functools.partial(
    pl.kernel,
    out_type=jax.ShapeDtypeStruct((OUT_ROWS, WDIM), jnp.float32),
    mesh=_mesh,
    scratch_types=[
        pltpu.VMEM((BATCH,), jnp.int32),
        pltpu.VMEM((BATCH + 16,), jnp.int32),
        pltpu.VMEM((BATCH + 16,), jnp.int32),
        pltpu.VMEM((DIM, CCOLS), jnp.float32),
        pltpu.VMEM((WINDOW + 16,), jnp.int32),
        pltpu.VMEM((WINDOW + 16,), jnp.int32),
        pltpu.VMEM((WINDOW // 128, 128), jnp.int32),
        pltpu.VMEM((WINDOW + 16, WDIM), jnp.float32),
        pltpu.VMEM((DIM, TAIL_COLS), jnp.float32),
        pltpu.SemaphoreType.DMA,
    ],
    compiler_params=pltpu.CompilerParams(use_tc_tiling_on_sc=True, needs_layout_passes=False),
)
def _sc_scan(idx_hbm, table_hbm, tail_hbm, out_hbm, idx_v, mcol_v, mpos_v,
             chunk_v, wl_loc, wl_pos, pos_w, outbuf, tail_v, sem):
    wid = lax.axis_index("s") * NUM_CORES + lax.axis_index("c")
    lo = wid * BASE_COLS + jnp.minimum(wid, EXTRA) * 128
    n_w = BASE_COLS + jnp.where(wid < EXTRA, 128, 0)
    is_tail_w = wid == NW - 1
    hi = jnp.where(is_tail_w, VOCAB, lo + n_w)
    last_off = n_w - CCOLS
    sink = BATCH + wid
    iota16 = lax.iota(jnp.int32, 16)

    pltpu.sync_copy(idx_hbm, idx_v)

    zeros16 = jnp.zeros((16,), jnp.int32)
    for i in range((WINDOW + 16) // 16):
        wl_loc[pl.ds(i * 16, 16)] = zeros16

    def _route(i, cnt):
        v = idx_v[pl.ds(i * 16, 16)]
        m = (v >= lo) & (v < hi)
        dst = cnt + plsc.cumsum(m.astype(jnp.int32)) - 1
        plsc.store_scatter(mcol_v, [dst], v - lo, mask=m)
        plsc.store_scatter(mpos_v, [dst], i * 16 + iota16, mask=m)
        return cnt + jnp.sum(m.astype(jnp.int32))

    cnt = lax.fori_loop(0, NIDX_G, _route, jnp.int32(0))
    mcol_v[pl.ds(cnt, 16)] = zeros16
    mpos_v[pl.ds(cnt, 16)] = jnp.full((16,), sink, jnp.int32)
    ng = (cnt + 15) >> 4

    def _process_block(src, off, cw):
        def _count(g, mc):
            cols = mcol_v[pl.ds(g * 16, 16)]
            valid = (g * 16 + iota16) < cnt
            m2 = valid & (cols >= off) & (cols < off + cw)
            return mc + jnp.sum(m2.astype(jnp.int32))

        mcnt = lax.fori_loop(0, ng, _count, jnp.int32(0))
        nwin = (mcnt + (WINDOW - 1)) // WINDOW

        def _window(s, _):
            base = s * WINDOW

            def _rescan(g, wc):
                cols = mcol_v[pl.ds(g * 16, 16)]
                pos = mpos_v[pl.ds(g * 16, 16)]
                valid = (g * 16 + iota16) < cnt
                m2 = valid & (cols >= off) & (cols < off + cw)
                pre = plsc.cumsum(m2.astype(jnp.int32)) + wc
                sel = m2 & (pre > base) & (pre <= base + WINDOW)
                dst = pre - 1 - base
                plsc.store_scatter(wl_loc, [dst], cols - off, mask=sel)
                plsc.store_scatter(wl_pos, [dst], pos, mask=sel)
                return wc + jnp.sum(m2.astype(jnp.int32))

            wc = lax.fori_loop(0, ng, _rescan, jnp.int32(0))
            wcn = jnp.minimum(wc - base, WINDOW)

            for i in range(WINDOW // 16):
                pv = wl_pos[pl.ds(i * 16, 16)]
                ok = (i * 16 + iota16) < wcn
                pos_w[i * 16 // 128, pl.ds((i * 16) % 128, 16)] = jnp.where(
                    ok, pv, jnp.full((16,), sink, jnp.int32))

            def _extract(e, _):
                lc = wl_loc[pl.ds(e * 16, 16)]
                slot = e * 16 + iota16
                for d in range(DIM):
                    vals = plsc.load_gather(
                        src, [jnp.full((16,), d, jnp.int32), lc])
                    plsc.store_scatter(
                        outbuf, [slot, jnp.full((16,), d, jnp.int32)], vals)
                return 0

            lax.fori_loop(0, (wcn + 15) >> 4, _extract, 0)

            cps = []
            for j in range(WINDOW // 128):
                cps.append(
                    pltpu.async_copy(
                        outbuf.at[pl.ds(j * 128, 128)],
                        out_hbm.at[pos_w.at[j]],
                        sem,
                    )
                )
            for cp in cps:
                cp.wait()
            return 0

        lax.fori_loop(0, nwin, _window, 0)

    _STAGE = 3

    def _chunk(k, _):
        off = jnp.minimum(k * CCOLS, last_off)
        pltpu.sync_copy(table_hbm.at[:, pl.ds(lo + off, CCOLS)], chunk_v)
        if _STAGE >= 2:
            _process_block(chunk_v, off, CCOLS)
        return 0

    if _STAGE >= 1:
        lax.fori_loop(0, NCHUNK, _chunk, 0)

    if _STAGE >= 3:
        @pl.when(is_tail_w)
        def _tail():
            pltpu.sync_copy(tail_hbm, tail_v)
            _process_block(tail_v, jnp.int32(BASE_COLS), TAIL_COLS)


_TC_ROWS = 2048


def _tc_narrow_body(wide_ref, out_ref):
    out_ref[...] = wide_ref[:, :DIM]


_tc_narrow = pl.pallas_call(
    _tc_narrow_body,
    grid=(BATCH // _TC_ROWS,),
    in_specs=[pl.BlockSpec((_TC_ROWS, WDIM), lambda i: (i, 0))],
    out_specs=pl.BlockSpec((_TC_ROWS, DIM), lambda i: (i, 0)),
    out_shape=jax.ShapeDtypeStruct((BATCH, DIM), jnp.float32),
)


def kernel(inputs, train_labels, embeddings):
    del train_labels
    table_t = embeddings.T
    tail_t = embeddings[TAIL_START:].T
    wide = _sc_scan(inputs, table_t, tail_t)
    return _tc_narrow(wide)


# speedup vs baseline: 2.4101x; 2.4101x over previous
import functools

import jax
import jax.numpy as jnp
from jax import lax
from jax.experimental import pallas as pl
from jax.experimental.pallas import tpu as pltpu
from jax.experimental.pallas import tpu_sc as plsc

VOCAB = 1000000
DIM = 64
WDIM = 128
BATCH = 16384

NUM_CORES = 2
NUM_SUBCORES = 16
NW = NUM_CORES * NUM_SUBCORES
BASE_COLS = 31232
EXTRA = 4
TAIL_START = 999936
TAIL_COLS = VOCAB - TAIL_START
CCOLS = 512
NPAIR = 31
WINDOW = 64
NIDX_G = BATCH // 16
OUT_ROWS = BATCH + NW
PACK_SHIFT = 15
COL_MASK = (1 << PACK_SHIFT) - 1

_mesh = plsc.VectorSubcoreMesh(core_axis_name="c", subcore_axis_name="s")


@functools.partial(
    pl.kernel,
    out_type=jax.ShapeDtypeStruct((OUT_ROWS, WDIM), jnp.float32),
    mesh=_mesh,
    scratch_types=[
        pltpu.VMEM((BATCH,), jnp.int32),
        pltpu.VMEM((BATCH + 16,), jnp.int32),
        pltpu.VMEM((DIM, CCOLS), jnp.float32),
        pltpu.VMEM((DIM, CCOLS), jnp.float32),
        pltpu.VMEM((WINDOW + 16,), jnp.int32),
        pltpu.VMEM((1, WINDOW), jnp.int32),
        pltpu.VMEM((WINDOW + 16, WDIM), jnp.float32),
        pltpu.SemaphoreType.DMA,
        pltpu.SemaphoreType.DMA,
        pltpu.SemaphoreType.DMA,
    ],
    compiler_params=pltpu.CompilerParams(
        use_tc_tiling_on_sc=True, needs_layout_passes=False),
)
def _sc_scan(idx_hbm, table_hbm, tail_hbm, out_hbm, idx_v, mp_v, ck_a, ck_b,
             wl_v, pos_w, outbuf, sem_a, sem_b, sem_s):
    wid = lax.axis_index("s") * NUM_CORES + lax.axis_index("c")
    lo = wid * BASE_COLS + jnp.minimum(wid, EXTRA) * 128
    n_w = BASE_COLS + jnp.where(wid < EXTRA, 128, 0)
    is_tail_w = wid == NW - 1
    hi = jnp.where(is_tail_w, VOCAB, lo + n_w)
    last_off = n_w - CCOLS
    sink = BATCH + wid
    iota16 = lax.iota(jnp.int32, 16)
    sink16 = jnp.full((16,), sink, jnp.int32)
    pad16 = jnp.full((16,), COL_MASK, jnp.int32) | (sink16 << PACK_SHIFT)
    wlpad16 = sink16 << PACK_SHIFT

    def off_k(k):
        return jnp.minimum(k * CCOLS, last_off)

    pltpu.sync_copy(idx_hbm, idx_v)

    for i in range((WINDOW + 16) // 16):
        wl_v[pl.ds(i * 16, 16)] = wlpad16

    def _route(i, cnt):
        v = idx_v[pl.ds(i * 16, 16)]
        m = (v >= lo) & (v < hi)
        packed = (v - lo) | ((i * 16 + iota16) << PACK_SHIFT)
        dst = cnt + plsc.cumsum(m.astype(jnp.int32)) - 1
        plsc.store_scatter(mp_v, [dst], packed, mask=m)
        return cnt + jnp.sum(m.astype(jnp.int32))

    cnt = lax.fori_loop(0, NIDX_G, _route, jnp.int32(0))
    mp_v[pl.ds(cnt, 16)] = pad16
    ng = (cnt + 15) >> 4

    def _window(src, off, cw, base):
        def _rescan(g, wc):
            e = mp_v[pl.ds(g * 16, 16)]
            col = e & COL_MASK
            m2 = (col >= off) & (col < off + cw)
            pre = plsc.cumsum(m2.astype(jnp.int32)) + wc
            sel = m2 & (pre > base) & (pre <= base + WINDOW)
            plsc.store_scatter(wl_v, [pre - 1 - base], e - off, mask=sel)
            return wc + jnp.sum(m2.astype(jnp.int32))

        wc = lax.fori_loop(0, ng, _rescan, jnp.int32(0))
        wcn = jnp.clip(wc - base, 0, WINDOW)
        wl_v[pl.ds(wcn, 16)] = wlpad16

        for i in range(WINDOW // 16):
            ew = wl_v[pl.ds(i * 16, 16)]
            ok = (i * 16 + iota16) < wcn
            pos_w[0, pl.ds(i * 16, 16)] = jnp.where(
                ok, lax.shift_right_logical(ew, PACK_SHIFT), sink16)

        def _extract(e, _):
            ew = wl_v[pl.ds(e * 16, 16)]
            lc = ew & COL_MASK
            slot = e * 16 + iota16
            for d in range(DIM):
                vals = plsc.load_gather(
                    src, [jnp.full((16,), d, jnp.int32), lc])
                plsc.store_scatter(
                    outbuf, [slot, jnp.full((16,), d, jnp.int32)], vals)
            return 0

        lax.fori_loop(0, (wcn + 15) >> 4, _extract, 0)

        pltpu.async_copy(outbuf.at[pl.ds(0, WINDOW)],
                         out_hbm.at[pos_w.at[0]], sem_s).wait()
        return wc

    def _process(src, off, cw):
        wc = _window(src, off, cw, jnp.int32(0))
        nwin = (wc + (WINDOW - 1)) >> 6

        def _more(s, _):
            _window(src, off, cw, s * WINDOW)
            return 0

        lax.fori_loop(1, nwin, _more, 0)

    def _pair(kk, _):
        k0 = kk * 2
        cpa = pltpu.async_copy(
            table_hbm.at[:, pl.ds(lo + off_k(k0), CCOLS)], ck_a, sem_a)
        cpb = pltpu.async_copy(
            table_hbm.at[:, pl.ds(lo + off_k(k0 + 1), CCOLS)], ck_b, sem_b)
        cpa.wait()
        _process(ck_a, off_k(k0), CCOLS)
        cpb.wait()
        _process(ck_b, off_k(k0 + 1), CCOLS)
        return 0

    lax.fori_loop(0, NPAIR, _pair, 0)

    @pl.when(is_tail_w)
    def _tail():
        pltpu.sync_copy(tail_hbm, ck_a.at[:, pl.ds(0, 128)])
        _process(ck_a, jnp.int32(BASE_COLS), TAIL_COLS)


_TC_ROWS = 2048


def _tc_narrow_body(wide_ref, out_ref):
    out_ref[...] = wide_ref[:, :DIM]


_tc_narrow = pl.pallas_call(
    _tc_narrow_body,
    grid=(BATCH // _TC_ROWS,),
    in_specs=[pl.BlockSpec((_TC_ROWS, WDIM), lambda i: (i, 0))],
    out_specs=pl.BlockSpec((_TC_ROWS, DIM), lambda i: (i, 0)),
    out_shape=jax.ShapeDtypeStruct((BATCH, DIM), jnp.float32),
)


def kernel(inputs, train_labels, embeddings):
    del train_labels
    table_t = embeddings.T
    tail_t = jnp.pad(embeddings[TAIL_START:], ((0, 128 - TAIL_COLS), (0, 0))).T
    wide = _sc_scan(inputs, table_t, tail_t)
    return _tc_narrow(wide)


# speedup vs baseline: 2.5337x; 1.0513x over previous
import functools

import jax
import jax.numpy as jnp
from jax import lax
from jax.experimental import pallas as pl
from jax.experimental.pallas import tpu as pltpu
from jax.experimental.pallas import tpu_sc as plsc

VOCAB = 1000000
DIM = 64
WDIM = 128
BATCH = 16384

NUM_CORES = 2
NUM_SUBCORES = 16
NW = NUM_CORES * NUM_SUBCORES
BASE_COLS = 31232
EXTRA = 4
TAIL_START = 999936
TAIL_COLS = VOCAB - TAIL_START
CCOLS = 512
NPAIR = 31
WINDOW = 64
NIDX_G = BATCH // 16
OUT_ROWS = BATCH + NW
PACK_SHIFT = 15
COL_MASK = (1 << PACK_SHIFT) - 1

_mesh = plsc.VectorSubcoreMesh(core_axis_name="c", subcore_axis_name="s")


@functools.partial(
    pl.kernel,
    out_type=jax.ShapeDtypeStruct((OUT_ROWS, WDIM), jnp.float32),
    mesh=_mesh,
    scratch_types=[
        pltpu.VMEM((BATCH,), jnp.int32),
        pltpu.VMEM((BATCH + 16,), jnp.int32),
        pltpu.VMEM((DIM, CCOLS), jnp.float32),
        pltpu.VMEM((DIM, CCOLS), jnp.float32),
        pltpu.VMEM((WINDOW + 16,), jnp.int32),
        pltpu.VMEM((1, WINDOW), jnp.int32),
        pltpu.VMEM((WINDOW + 16, WDIM), jnp.float32),
        pltpu.SemaphoreType.DMA,
        pltpu.SemaphoreType.DMA,
        pltpu.SemaphoreType.DMA,
    ],
    compiler_params=pltpu.CompilerParams(
        use_tc_tiling_on_sc=True, needs_layout_passes=False),
)
def _sc_scan(idx_hbm, table_hbm, tail_hbm, out_hbm, idx_v, mp_v, ck_a, ck_b,
             wl_v, pos_w, outbuf, sem_a, sem_b, sem_s):
    wid = lax.axis_index("s") * NUM_CORES + lax.axis_index("c")
    lo = wid * BASE_COLS + jnp.minimum(wid, EXTRA) * 128
    n_w = BASE_COLS + jnp.where(wid < EXTRA, 128, 0)
    is_tail_w = wid == NW - 1
    hi = jnp.where(is_tail_w, VOCAB, lo + n_w)
    last_off = n_w - CCOLS
    sink = BATCH + wid
    iota16 = lax.iota(jnp.int32, 16)
    sink16 = jnp.full((16,), sink, jnp.int32)
    pad16 = jnp.full((16,), COL_MASK, jnp.int32) | (sink16 << PACK_SHIFT)
    wlpad16 = sink16 << PACK_SHIFT

    def off_k(k):
        return jnp.minimum(k * CCOLS, last_off)

    pltpu.sync_copy(idx_hbm, idx_v)

    for i in range((WINDOW + 16) // 16):
        wl_v[pl.ds(i * 16, 16)] = wlpad16
    for i in range(WINDOW // 16):
        pos_w[0, pl.ds(i * 16, 16)] = sink16
    pltpu.async_copy(outbuf.at[pl.ds(0, WINDOW)], out_hbm.at[pos_w.at[0]],
                     sem_s)

    def _route(i, cnt):
        v = idx_v[pl.ds(i * 16, 16)]
        m = (v >= lo) & (v < hi)
        packed = (v - lo) | ((i * 16 + iota16) << PACK_SHIFT)
        pre = cnt + plsc.cumsum(m.astype(jnp.int32))
        plsc.store_scatter(mp_v, [pre - 1], packed, mask=m)
        return pre[15]

    cnt = lax.fori_loop(0, NIDX_G, _route, jnp.int32(0))
    mp_v[pl.ds(cnt, 16)] = pad16
    ng = (cnt + 15) >> 4

    def _window(src, off, cw, base):
        def _rescan(g, wc):
            e = mp_v[pl.ds(g * 16, 16)]
            col = e & COL_MASK
            m2 = (col >= off) & (col < off + cw)
            pre = plsc.cumsum(m2.astype(jnp.int32)) + wc
            sel = m2 & (pre > base) & (pre <= base + WINDOW)
            plsc.store_scatter(wl_v, [pre - 1 - base], e - off, mask=sel)
            return pre[15]

        wc = lax.fori_loop(0, ng, _rescan, jnp.int32(0))
        wcn = jnp.clip(wc - base, 0, WINDOW)
        wl_v[pl.ds(wcn, 16)] = wlpad16

        pltpu.make_async_copy(out_hbm.at[pl.ds(0, WINDOW)],
                              outbuf.at[pl.ds(0, WINDOW)], sem_s).wait()

        for i in range(WINDOW // 16):
            ew = wl_v[pl.ds(i * 16, 16)]
            ok = (i * 16 + iota16) < wcn
            pos_w[0, pl.ds(i * 16, 16)] = jnp.where(
                ok, lax.shift_right_logical(ew, PACK_SHIFT), sink16)

        def _extract(e, _):
            ew = wl_v[pl.ds(e * 16, 16)]
            lc = ew & COL_MASK
            slot = e * 16 + iota16
            for d in range(DIM):
                vals = plsc.load_gather(
                    src, [jnp.full((16,), d, jnp.int32), lc])
                plsc.store_scatter(
                    outbuf, [slot, jnp.full((16,), d, jnp.int32)], vals)
            return 0

        lax.fori_loop(0, (wcn + 15) >> 4, _extract, 0)

        pltpu.async_copy(outbuf.at[pl.ds(0, WINDOW)],
                         out_hbm.at[pos_w.at[0]], sem_s)
        return wc

    def _process(src, off, cw):
        wc = _window(src, off, cw, jnp.int32(0))
        nwin = (wc + (WINDOW - 1)) >> 6

        def _more(s, _):
            _window(src, off, cw, s * WINDOW)
            return 0

        lax.fori_loop(1, nwin, _more, 0)

    def _pair(kk, _):
        k0 = kk * 2
        cpa = pltpu.async_copy(
            table_hbm.at[:, pl.ds(lo + off_k(k0), CCOLS)], ck_a, sem_a)
        cpb = pltpu.async_copy(
            table_hbm.at[:, pl.ds(lo + off_k(k0 + 1), CCOLS)], ck_b, sem_b)
        cpa.wait()
        _process(ck_a, off_k(k0), CCOLS)
        cpb.wait()
        _process(ck_b, off_k(k0 + 1), CCOLS)
        return 0

    lax.fori_loop(0, NPAIR, _pair, 0)

    @pl.when(is_tail_w)
    def _tail():
        pltpu.sync_copy(tail_hbm, ck_a.at[:, pl.ds(0, 128)])
        _process(ck_a, jnp.int32(BASE_COLS), TAIL_COLS)

    pltpu.make_async_copy(out_hbm.at[pl.ds(0, WINDOW)],
                          outbuf.at[pl.ds(0, WINDOW)], sem_s).wait()


_TC_ROWS = 2048


def _tc_narrow_body(wide_ref, out_ref):
    out_ref[...] = wide_ref[:, :DIM]


_tc_narrow = pl.pallas_call(
    _tc_narrow_body,
    grid=(BATCH // _TC_ROWS,),
    in_specs=[pl.BlockSpec((_TC_ROWS, WDIM), lambda i: (i, 0))],
    out_specs=pl.BlockSpec((_TC_ROWS, DIM), lambda i: (i, 0)),
    out_shape=jax.ShapeDtypeStruct((BATCH, DIM), jnp.float32),
)


def kernel(inputs, train_labels, embeddings):
    del train_labels
    table_t = embeddings.T
    tail_t = jnp.pad(embeddings[TAIL_START:], ((0, 128 - TAIL_COLS), (0, 0))).T
    wide = _sc_scan(inputs, table_t, tail_t)
    return _tc_narrow(wide)
